# baseline (speedup 1.0000x reference)
import functools

import jax
import jax.numpy as jnp
from jax import lax
from jax.experimental import pallas as pl
from jax.experimental.pallas import tpu as pltpu
from jax.experimental.pallas import tpu_sc as plsc

_T = 32768
_E = 8
_K = 2
_BLK = 1024
_L = 16
_NW = 32
_CHUNK = _T // _NW


def _logits_kernel(h_ref, gw_ref, lt_ref):
    h = h_ref[...]
    gw = gw_ref[...]
    logits = jax.lax.dot_general(
        h.astype(jnp.bfloat16), gw.astype(jnp.bfloat16),
        (((1,), (1,)), ((), ())),
        preferred_element_type=jnp.float32,
    )
    lt_ref[...] = jax.lax.transpose(logits, (1, 0))


def _tc_logits(hidden_states, gate_w):
    t = hidden_states.shape[0]
    e = gate_w.shape[0]
    grid = t // _BLK
    return pl.pallas_call(
        _logits_kernel,
        grid=(grid,),
        in_specs=[
            pl.BlockSpec((_BLK, hidden_states.shape[1]), lambda i: (i, 0)),
            pl.BlockSpec((e, hidden_states.shape[1]), lambda i: (0, 0)),
        ],
        out_specs=pl.BlockSpec((e, _BLK), lambda i: (0, i)),
        out_shape=jax.ShapeDtypeStruct((e, t), jnp.float32),
        compiler_params=pltpu.CompilerParams(
            dimension_semantics=("arbitrary",),
        ),
    )(hidden_states, gate_w)


def _sc_route(lt_hbm, bias_hbm, p1_hbm, pk_hbm, lg_v, bias_v, p1_v, pk_v):
    wid = lax.axis_index("s") * 2 + lax.axis_index("c")
    base = wid * _CHUNK
    pltpu.sync_copy(lt_hbm.at[:, pl.ds(base, _CHUNK)], lg_v)
    pltpu.sync_copy(bias_hbm, bias_v)

    def body(j, _):
        off = j * _L
        l = [lg_v[e, pl.ds(off, _L)] for e in range(_E)]
        m = l[0]
        for e in range(1, _E):
            m = jnp.maximum(m, l[e])
        ex = [jnp.exp(l[e] - m) for e in range(_E)]
        s = ex[0]
        for e in range(1, _E):
            s = s + ex[e]
        sc = [ex[e] / s for e in range(_E)]
        sel = [sc[e] + bias_v[e, :] for e in range(_E)]

        econst = [jnp.full((_L,), e, jnp.int32) for e in range(_E)]
        m1 = sel[0]
        for e in range(1, _E):
            m1 = jnp.maximum(m1, sel[e])
        i1 = jnp.full((_L,), _E, jnp.int32)
        for e in range(_E - 1, -1, -1):
            i1 = jnp.where(sel[e] == m1, econst[e], i1)
        ninf = jnp.full((_L,), -jnp.inf, jnp.float32)
        sel2 = [jnp.where(i1 == econst[e], ninf, sel[e]) for e in range(_E)]
        m2 = sel2[0]
        for e in range(1, _E):
            m2 = jnp.maximum(m2, sel2[e])
        i2 = jnp.full((_L,), _E, jnp.int32)
        for e in range(_E - 1, -1, -1):
            i2 = jnp.where(sel2[e] == m2, econst[e], i2)

        zero = jnp.zeros((_L,), jnp.float32)
        p1 = zero
        p2 = zero
        bits = jnp.zeros((_L,), jnp.int32)
        for e in range(_E):
            sel_e1 = i1 == econst[e]
            sel_e2 = i2 == econst[e]
            p1 = p1 + jnp.where(sel_e1, sc[e], zero)
            p2 = p2 + jnp.where(sel_e2, sc[e], zero)
            bits = bits | jnp.where(
                sel_e1 | sel_e2,
                jnp.full((_L,), 1 << e, jnp.int32),
                jnp.zeros((_L,), jnp.int32))
        p1_v[pl.ds(off, _L)] = p1 / (p1 + p2 + 1e-9)
        pk_v[pl.ds(off, _L)] = i1 | (i2 << 3) | (bits << 6)
        return _

    lax.fori_loop(0, _CHUNK // _L, body, None)
    pltpu.sync_copy(p1_v, p1_hbm.at[pl.ds(base, _CHUNK)])
    pltpu.sync_copy(pk_v, pk_hbm.at[pl.ds(base, _CHUNK)])


@jax.jit
def kernel(hidden_states, gate_w, expert_bias):
    t = hidden_states.shape[0]
    e = gate_w.shape[0]
    lt = _tc_logits(hidden_states, gate_w)
    bias_b = jnp.tile(expert_bias.reshape(e, 1), (1, _L))
    mesh = plsc.VectorSubcoreMesh(core_axis_name="c", subcore_axis_name="s")
    p1_row, pk_row = pl.kernel(
        _sc_route,
        mesh=mesh,
        out_type=[
            jax.ShapeDtypeStruct((t,), jnp.float32),
            jax.ShapeDtypeStruct((t,), jnp.int32),
        ],
        scratch_types=[
            pltpu.VMEM((e, _CHUNK), jnp.float32),
            pltpu.VMEM((e, _L), jnp.float32),
            pltpu.VMEM((_CHUNK,), jnp.float32),
            pltpu.VMEM((_CHUNK,), jnp.int32),
        ],
    )(lt, bias_b)
    p1 = p1_row
    v = pk_row
    probs = jnp.stack([p1, 1.0 - p1], axis=-1)
    idx = jnp.stack([v & 7, (v >> 3) & 7], axis=-1)
    rmap = ((v[:, None] >> (jnp.arange(_E, dtype=jnp.int32) + 6)) & 1) != 0
    aux_loss = jnp.zeros((), dtype=jnp.float32)
    return probs, idx, rmap, aux_loss
